# baseline (speedup 1.0000x reference)
import functools
import math

import jax
import jax.numpy as jnp
from jax import lax
from jax.experimental import pallas as pl
from jax.experimental.pallas import tpu as pltpu
from jax.experimental.pallas import tpu_sc as plsc

VOCAB = 32000
SMOOTH = 0.1
CONF = 1.0 - SMOOTH
SV = SMOOTH / (VOCAB - 2 + 1e-06)
LOG_SV = math.log(SV)
LOG_CONF = math.log(CONF)
ENT_BASE = (VOCAB - 2) * SV * LOG_SV
ENT_DELTA = CONF * LOG_CONF - SV * LOG_SV

RB = 1024
VB = 3200

NC, NS, LANES = 2, 16, 16
NW = NC * NS


def _gather_body(x_hbm, tgt_hbm, out_hbm, tgt_v, idx_v, val_v, sem, *, bpw):
    wid = lax.axis_index("s") * NC + lax.axis_index("c")
    base = wid * bpw
    pltpu.sync_copy(tgt_hbm.at[pl.ds(base, bpw)], tgt_v)
    for k in range(bpw // LANES):
        t16 = tgt_v[pl.ds(k * LANES, LANES)]
        rows = (base + k * LANES) + lax.broadcasted_iota(jnp.int32, (LANES,), 0)
        idx_v[pl.ds(k * LANES, LANES)] = rows * VOCAB + t16
    pltpu.async_copy(x_hbm.at[idx_v], val_v, sem).wait()
    pltpu.sync_copy(val_v, out_hbm.at[pl.ds(base, bpw)])


def _sc_gather(x_flat, tgt1d):
    rows = tgt1d.shape[0]
    bpw = rows // NW
    mesh = plsc.VectorSubcoreMesh(core_axis_name="c", subcore_axis_name="s")
    return pl.kernel(
        functools.partial(_gather_body, bpw=bpw),
        out_type=jax.ShapeDtypeStruct((rows,), jnp.float32),
        mesh=mesh,
        scratch_types=[
            pltpu.VMEM((bpw,), jnp.int32),
            pltpu.VMEM((bpw,), jnp.int32),
            pltpu.VMEM((bpw,), jnp.float32),
            pltpu.SemaphoreType.DMA,
        ],
    )(x_flat, tgt1d)


def _loss_body(x_ref, tgt_ref, xt_ref, out_ref):
    i = pl.program_id(0)
    j = pl.program_id(1)
    nr = pl.num_programs(0)
    nv = pl.num_programs(1)

    @pl.when((i == 0) & (j == 0))
    def _init():
        out_ref[...] = jnp.zeros_like(out_ref)

    blk = x_ref[...]
    acc = -SV * jnp.sum(blk)

    @pl.when(j == 0)
    def _edge():
        out_ref[...] = out_ref[...] + SV * jnp.sum(blk[:, 0] + blk[:, 1])

    @pl.when((i == nr - 1) & (j == nv - 1))
    def _epilogue():
        t_all = tgt_ref[0, :]
        m = t_all >= 2
        cnt = jnp.sum(jnp.where(m, 1.0, 0.0))
        xt_hit = jnp.sum(jnp.where(m, xt_ref[0, :], 0.0))
        out_ref[...] = (out_ref[...]
                        + (t_all.shape[0] * ENT_BASE + cnt * ENT_DELTA)
                        - (CONF - SV) * xt_hit)

    out_ref[...] = out_ref[...] + acc


def _loss_call(x2d, tgt2d, xt2d):
    rows = x2d.shape[0]
    nr = rows // RB
    nv = VOCAB // VB
    out = pl.pallas_call(
        _loss_body,
        grid=(nr, nv),
        in_specs=[
            pl.BlockSpec((RB, VB), lambda i, j: (i, j)),
            pl.BlockSpec((1, rows), lambda i, j: (0, 0)),
            pl.BlockSpec((1, rows), lambda i, j: (0, 0)),
        ],
        out_specs=pl.BlockSpec((1, 1), lambda i, j: (0, 0)),
        out_shape=jax.ShapeDtypeStruct((1, 1), jnp.float32),
    )(x2d, tgt2d, xt2d)
    return out[0, 0]


def kernel(x, target):
    rows = x.shape[0] * x.shape[1]
    x2d = x.reshape(rows, VOCAB)
    tgt1d = target.reshape(rows)
    xt = _sc_gather(x2d.reshape(rows * VOCAB), tgt1d)
    return _loss_call(x2d, tgt1d.reshape(1, rows), xt.reshape(1, rows))


# speedup vs baseline: 3.0561x; 3.0561x over previous
import functools
import math

import jax
import jax.numpy as jnp
from jax.experimental import pallas as pl
from jax.experimental.pallas import tpu as pltpu

VOCAB = 32000
SMOOTH = 0.1
CONF = 1.0 - SMOOTH
SV = SMOOTH / (VOCAB - 2 + 1e-06)
LOG_SV = math.log(SV)
LOG_CONF = math.log(CONF)
ENT_BASE = (VOCAB - 2) * SV * LOG_SV
ENT_DELTA = CONF * LOG_CONF - SV * LOG_SV

RB = 1024
VB = 3200


def _loss_body(x_ref, tgt_ref, out_ref):
    i = pl.program_id(0)
    j = pl.program_id(1)
    nr = pl.num_programs(0)
    nv = pl.num_programs(1)

    @pl.when((i == 0) & (j == 0))
    def _init():
        out_ref[...] = jnp.zeros_like(out_ref)

    blk = x_ref[...]
    tgt = tgt_ref[0, pl.ds(i * RB, RB)]
    tloc = (tgt - j * VB)[:, None]
    scale = jnp.where(tgt[:, None] >= 2, CONF / SV, 1.0)
    cols = jax.lax.broadcasted_iota(jnp.int32, (RB, VB), 1)
    val = jnp.where(cols == tloc, blk * scale, blk)

    acc = -SV * jnp.sum(val)

    @pl.when(j == 0)
    def _edge():
        out_ref[...] = out_ref[...] + SV * jnp.sum(blk[:, 0] + blk[:, 1])

    @pl.when((i == nr - 1) & (j == nv - 1))
    def _entropy():
        t_all = tgt_ref[0, :]
        cnt = jnp.sum(jnp.where(t_all >= 2, 1.0, 0.0))
        out_ref[...] = out_ref[...] + (t_all.shape[0] * ENT_BASE + cnt * ENT_DELTA)

    out_ref[...] = out_ref[...] + acc


def _loss_call(x2d, tgt2d, interpret=False):
    rows = x2d.shape[0]
    nr = rows // RB
    nv = VOCAB // VB
    out = pl.pallas_call(
        _loss_body,
        grid=(nr, nv),
        in_specs=[
            pl.BlockSpec((RB, VB), lambda i, j: (i, j)),
            pl.BlockSpec((1, rows), lambda i, j: (0, 0)),
        ],
        out_specs=pl.BlockSpec((1, 1), lambda i, j: (0, 0)),
        out_shape=jax.ShapeDtypeStruct((1, 1), jnp.float32),
        interpret=interpret,
    )(x2d, tgt2d)
    return out[0, 0]


def kernel(x, target):
    rows = x.shape[0] * x.shape[1]
    x2d = x.reshape(rows, VOCAB)
    tgt2d = target.reshape(1, rows)
    return _loss_call(x2d, tgt2d)
